# baseline (speedup 1.0000x reference)
import jax
import jax.numpy as jnp
from jax.experimental import pallas as pl
from jax.experimental.pallas import tpu as pltpu

SEQ = 512
BAT = 64
HID = 768
T = 32
TT = T * T
START = 30
END = 31
BS = 8
NBLK = SEQ // BS
ROWS = BS * BAT
NCH = 4
HB = BAT // NCH
LOG_T = 3.4657359027997265


def _phase(k, feats_ref, tgt_ref, mskp_ref, mskc_ref, wt_ref, b_ref, e_ref,
           s_ref, ex_prod, bnd_prod, ex_cons, bnd_cons, p_ref, o_ref, tg_ref):
    fb = feats_ref[...].astype(jnp.bfloat16)
    scores = (jnp.dot(fb, wt_ref[...], preferred_element_type=jnp.float32)
              + b_ref[...])
    lane = jax.lax.broadcasted_iota(jnp.int32, (BAT, TT), 1)
    tgt2 = tgt_ref[0]
    mskp = mskp_ref[0]
    tg = jnp.where(k == 0, 0.0, tg_ref[0, 0])
    tgblk = 0.0
    for i in range(BS):
        sc = jax.lax.slice(scores, (i * BAT, 0), ((i + 1) * BAT, TT))
        tcol = jax.lax.slice(tgt2, (0, i), (BAT, i + 1))
        mcol = jax.lax.slice(mskp, (0, i), (BAT, i + 1))
        tgblk = tgblk + jnp.sum(jnp.where((lane == tcol) & (mcol > 0.0), sc, 0.0))
        bound = jnp.max(sc, axis=1, keepdims=True) + LOG_T
        ex_prod[i * BAT:(i + 1) * BAT, :] = jnp.exp(sc - bound)
        bnd_prod[:, i:i + 1] = bound
    tg_ref[0, 0] = tg + jnp.where(k < NBLK, tgblk, 0.0)

    mskc = mskc_ref[0]
    e = e_ref[...]
    s = s_ref[...]
    P = [p_ref[c * HB:(c + 1) * HB, :] for c in range(NCH)]
    O = [o_ref[c * HB:(c + 1) * HB, :] for c in range(NCH)]
    for i in range(BS):
        exi = [ex_cons[i * BAT + c * HB:i * BAT + (c + 1) * HB, :]
               for c in range(NCH)]
        g = [jnp.dot(P[c], e, preferred_element_type=jnp.float32)
             for c in range(NCH)]
        ex = [g[c] * exi[c] for c in range(NCH)]
        pn = [jnp.dot(ex[c], s, preferred_element_type=jnp.float32)
              for c in range(NCH)]
        bcol = bnd_cons[:, i:i + 1]
        mcol = jax.lax.slice(mskc, (0, i), (BAT, i + 1)) > 0.0
        for c in range(NCH):
            lo = c * HB
            mc = jax.lax.slice(mcol, (lo, 0), (lo + HB, 1))
            bc = jax.lax.slice(bcol, (lo, 0), (lo + HB, 1))
            pnew = jnp.where(mc, pn[c], P[c])
            onew = jnp.where(mc, O[c] + bc, O[c])
            if i == 0:
                init_p = ex_cons[lo:lo + HB, START * T:START * T + T]
                pnew = jnp.where(k == 1, init_p, pnew)
                onew = jnp.where(k == 1, jnp.zeros_like(onew) + bc, onew)
            P[c], O[c] = pnew, onew
    for c in range(NCH):
        lo = c * HB
        pmax = jnp.maximum(jnp.max(P[c], axis=1, keepdims=True), 1e-30)
        p_ref[lo:lo + HB, :] = P[c] / pmax
        o_ref[lo:lo + HB, :] = O[c] + jnp.log(pmax)


def _crf_body(feats_ref, tgt_ref, mskp_ref, mskc_ref, wt_ref, b_ref, e_ref,
              s_ref, out_ref, ex_a, ex_b, bnd_a, bnd_b, p_ref, o_ref, tg_ref):
    k = pl.program_id(0)
    p = jax.lax.rem(k, 2)

    @pl.when(p == 0)
    def _():
        _phase(k, feats_ref, tgt_ref, mskp_ref, mskc_ref, wt_ref, b_ref,
               e_ref, s_ref, ex_a, bnd_a, ex_b, bnd_b, p_ref, o_ref, tg_ref)

    @pl.when(p == 1)
    def _():
        _phase(k, feats_ref, tgt_ref, mskp_ref, mskc_ref, wt_ref, b_ref,
               e_ref, s_ref, ex_b, bnd_b, ex_a, bnd_a, p_ref, o_ref, tg_ref)

    @pl.when(k == NBLK)
    def _():
        pend = jnp.log(jnp.maximum(p_ref[...], 1e-38)) + o_ref[...]
        logz = jnp.sum(jax.lax.slice(pend, (0, END), (BAT, END + 1)))
        out_ref[0, 0] = (logz - tg_ref[0, 0]) / float(BAT)


def kernel(feats, target, mask, W, b):
    feats2 = feats.reshape(SEQ * BAT, HID)
    wt = W.T.astype(jnp.bfloat16)
    b2 = b.reshape(1, TT)
    tgt = target[..., 0].astype(jnp.int32).reshape(NBLK, BS, BAT).transpose(0, 2, 1)
    msk = mask.astype(jnp.float32).reshape(NBLK, BS, BAT).transpose(0, 2, 1)
    jj = jnp.arange(TT, dtype=jnp.int32)
    e_mat = (jj[None, :] // T == jnp.arange(T, dtype=jnp.int32)[:, None]).astype(jnp.float32)
    s_mat = (jj[:, None] % T == jnp.arange(T, dtype=jnp.int32)[None, :]).astype(jnp.float32)

    out = pl.pallas_call(
        _crf_body,
        grid=(NBLK + 1,),
        in_specs=[
            pl.BlockSpec((ROWS, HID), lambda k: (jnp.minimum(k, NBLK - 1), 0)),
            pl.BlockSpec((1, BAT, BS), lambda k: (jnp.minimum(k, NBLK - 1), 0, 0)),
            pl.BlockSpec((1, BAT, BS), lambda k: (jnp.minimum(k, NBLK - 1), 0, 0)),
            pl.BlockSpec((1, BAT, BS), lambda k: (jnp.maximum(k - 1, 0), 0, 0)),
            pl.BlockSpec((HID, TT), lambda k: (0, 0)),
            pl.BlockSpec((1, TT), lambda k: (0, 0)),
            pl.BlockSpec((T, TT), lambda k: (0, 0)),
            pl.BlockSpec((TT, T), lambda k: (0, 0)),
        ],
        out_specs=pl.BlockSpec((1, 1), lambda k: (0, 0), memory_space=pltpu.SMEM),
        out_shape=jax.ShapeDtypeStruct((1, 1), jnp.float32),
        scratch_shapes=[
            pltpu.VMEM((ROWS, TT), jnp.float32),
            pltpu.VMEM((ROWS, TT), jnp.float32),
            pltpu.VMEM((BAT, BS), jnp.float32),
            pltpu.VMEM((BAT, BS), jnp.float32),
            pltpu.VMEM((BAT, T), jnp.float32),
            pltpu.VMEM((BAT, T), jnp.float32),
            pltpu.SMEM((1, 1), jnp.float32),
        ],
        compiler_params=pltpu.CompilerParams(dimension_semantics=("arbitrary",)),
    )(feats2, tgt, msk, msk, wt, b2, e_mat, s_mat)
    return out[0, 0]


# speedup vs baseline: 1.0527x; 1.0527x over previous
import jax
import jax.numpy as jnp
from jax.experimental import pallas as pl
from jax.experimental.pallas import tpu as pltpu

SEQ = 512
BAT = 64
HID = 768
T = 32
TT = T * T
START = 30
END = 31
BS = 16
NBLK = SEQ // BS
ROWS = BS * BAT
NCH = 4
HB = BAT // NCH
LOG_T = 3.4657359027997265


def _phase(k, feats_ref, tgt_ref, mskp_ref, mskc_ref, wt_ref, b_ref, e_ref,
           s_ref, ex_prod, bnd_prod, ex_cons, bnd_cons, p_ref, o_ref, tg_ref):
    fb = feats_ref[...].astype(jnp.bfloat16)
    scores = (jnp.dot(fb, wt_ref[...], preferred_element_type=jnp.float32)
              + b_ref[...])
    lane = jax.lax.broadcasted_iota(jnp.int32, (BAT, TT), 1)
    tgt2 = tgt_ref[0]
    mskp = mskp_ref[0]
    tg = jnp.where(k == 0, 0.0, tg_ref[0, 0])
    tgblk = 0.0
    for i in range(BS):
        sc = jax.lax.slice(scores, (i * BAT, 0), ((i + 1) * BAT, TT))
        tcol = jax.lax.slice(tgt2, (0, i), (BAT, i + 1))
        mcol = jax.lax.slice(mskp, (0, i), (BAT, i + 1))
        tgblk = tgblk + jnp.sum(jnp.where((lane == tcol) & (mcol > 0.0), sc, 0.0))
        bound = jnp.max(sc, axis=1, keepdims=True) + LOG_T
        ex_prod[i * BAT:(i + 1) * BAT, :] = jnp.exp(sc - bound)
        bnd_prod[:, i:i + 1] = bound
    tg_ref[0, 0] = tg + jnp.where(k < NBLK, tgblk, 0.0)

    mskc = mskc_ref[0]
    e = e_ref[...]
    s = s_ref[...]
    P = [p_ref[c * HB:(c + 1) * HB, :] for c in range(NCH)]
    O = [o_ref[c * HB:(c + 1) * HB, :] for c in range(NCH)]
    for i in range(BS):
        exi = [ex_cons[i * BAT + c * HB:i * BAT + (c + 1) * HB, :]
               for c in range(NCH)]
        g = [jnp.dot(P[c], e, preferred_element_type=jnp.float32)
             for c in range(NCH)]
        ex = [g[c] * exi[c] for c in range(NCH)]
        pn = [jnp.dot(ex[c], s, preferred_element_type=jnp.float32)
              for c in range(NCH)]
        bcol = bnd_cons[:, i:i + 1]
        mcol = jax.lax.slice(mskc, (0, i), (BAT, i + 1)) > 0.0
        for c in range(NCH):
            lo = c * HB
            mc = jax.lax.slice(mcol, (lo, 0), (lo + HB, 1))
            bc = jax.lax.slice(bcol, (lo, 0), (lo + HB, 1))
            pnew = jnp.where(mc, pn[c], P[c])
            onew = jnp.where(mc, O[c] + bc, O[c])
            if i == 0:
                init_p = ex_cons[lo:lo + HB, START * T:START * T + T]
                pnew = jnp.where(k == 1, init_p, pnew)
                onew = jnp.where(k == 1, jnp.zeros_like(onew) + bc, onew)
            P[c], O[c] = pnew, onew
    for c in range(NCH):
        lo = c * HB
        pmax = jnp.maximum(jnp.max(P[c], axis=1, keepdims=True), 1e-30)
        p_ref[lo:lo + HB, :] = P[c] / pmax
        o_ref[lo:lo + HB, :] = O[c] + jnp.log(pmax)


def _crf_body(feats_ref, tgt_ref, mskp_ref, mskc_ref, wt_ref, b_ref, e_ref,
              s_ref, out_ref, ex_a, ex_b, bnd_a, bnd_b, p_ref, o_ref, tg_ref):
    k = pl.program_id(0)
    p = jax.lax.rem(k, 2)

    @pl.when(p == 0)
    def _():
        _phase(k, feats_ref, tgt_ref, mskp_ref, mskc_ref, wt_ref, b_ref,
               e_ref, s_ref, ex_a, bnd_a, ex_b, bnd_b, p_ref, o_ref, tg_ref)

    @pl.when(p == 1)
    def _():
        _phase(k, feats_ref, tgt_ref, mskp_ref, mskc_ref, wt_ref, b_ref,
               e_ref, s_ref, ex_b, bnd_b, ex_a, bnd_a, p_ref, o_ref, tg_ref)

    @pl.when(k == NBLK)
    def _():
        pend = jnp.log(jnp.maximum(p_ref[...], 1e-38)) + o_ref[...]
        logz = jnp.sum(jax.lax.slice(pend, (0, END), (BAT, END + 1)))
        out_ref[0, 0] = (logz - tg_ref[0, 0]) / float(BAT)


def kernel(feats, target, mask, W, b):
    feats2 = feats.reshape(SEQ * BAT, HID)
    wt = W.T.astype(jnp.bfloat16)
    b2 = b.reshape(1, TT)
    tgt = target[..., 0].astype(jnp.int32).reshape(NBLK, BS, BAT).transpose(0, 2, 1)
    msk = mask.astype(jnp.float32).reshape(NBLK, BS, BAT).transpose(0, 2, 1)
    jj = jnp.arange(TT, dtype=jnp.int32)
    e_mat = (jj[None, :] // T == jnp.arange(T, dtype=jnp.int32)[:, None]).astype(jnp.float32)
    s_mat = (jj[:, None] % T == jnp.arange(T, dtype=jnp.int32)[None, :]).astype(jnp.float32)

    out = pl.pallas_call(
        _crf_body,
        grid=(NBLK + 1,),
        in_specs=[
            pl.BlockSpec((ROWS, HID), lambda k: (jnp.minimum(k, NBLK - 1), 0)),
            pl.BlockSpec((1, BAT, BS), lambda k: (jnp.minimum(k, NBLK - 1), 0, 0)),
            pl.BlockSpec((1, BAT, BS), lambda k: (jnp.minimum(k, NBLK - 1), 0, 0)),
            pl.BlockSpec((1, BAT, BS), lambda k: (jnp.maximum(k - 1, 0), 0, 0)),
            pl.BlockSpec((HID, TT), lambda k: (0, 0)),
            pl.BlockSpec((1, TT), lambda k: (0, 0)),
            pl.BlockSpec((T, TT), lambda k: (0, 0)),
            pl.BlockSpec((TT, T), lambda k: (0, 0)),
        ],
        out_specs=pl.BlockSpec((1, 1), lambda k: (0, 0), memory_space=pltpu.SMEM),
        out_shape=jax.ShapeDtypeStruct((1, 1), jnp.float32),
        scratch_shapes=[
            pltpu.VMEM((ROWS, TT), jnp.float32),
            pltpu.VMEM((ROWS, TT), jnp.float32),
            pltpu.VMEM((BAT, BS), jnp.float32),
            pltpu.VMEM((BAT, BS), jnp.float32),
            pltpu.VMEM((BAT, T), jnp.float32),
            pltpu.VMEM((BAT, T), jnp.float32),
            pltpu.SMEM((1, 1), jnp.float32),
        ],
        compiler_params=pltpu.CompilerParams(dimension_semantics=("arbitrary",)),
    )(feats2, tgt, msk, msk, wt, b2, e_mat, s_mat)
    return out[0, 0]


# speedup vs baseline: 1.0627x; 1.0095x over previous
import jax
import jax.numpy as jnp
from jax.experimental import pallas as pl
from jax.experimental.pallas import tpu as pltpu

SEQ = 512
BAT = 64
HID = 768
T = 32
TT = T * T
START = 30
END = 31
BS = 32
NBLK = SEQ // BS
ROWS = BS * BAT
NCH = 4
HB = BAT // NCH
LOG_T = 3.4657359027997265


def _phase(k, feats_ref, tgt_ref, mskp_ref, mskc_ref, wt_ref, b_ref, e_ref,
           s_ref, ex_prod, bnd_prod, ex_cons, bnd_cons, p_ref, o_ref, tg_ref):
    fb = feats_ref[...].astype(jnp.bfloat16)
    scores = (jnp.dot(fb, wt_ref[...], preferred_element_type=jnp.float32)
              + b_ref[...])
    lane = jax.lax.broadcasted_iota(jnp.int32, (BAT, TT), 1)
    tgt2 = tgt_ref[0]
    mskp = mskp_ref[0]
    tg = jnp.where(k == 0, 0.0, tg_ref[0, 0])
    tgblk = 0.0
    for i in range(BS):
        sc = jax.lax.slice(scores, (i * BAT, 0), ((i + 1) * BAT, TT))
        tcol = jax.lax.slice(tgt2, (0, i), (BAT, i + 1))
        mcol = jax.lax.slice(mskp, (0, i), (BAT, i + 1))
        tgblk = tgblk + jnp.sum(jnp.where((lane == tcol) & (mcol > 0.0), sc, 0.0))
        bound = jnp.max(sc, axis=1, keepdims=True) + LOG_T
        ex_prod[i * BAT:(i + 1) * BAT, :] = jnp.exp(sc - bound)
        bnd_prod[:, i:i + 1] = bound
    tg_ref[0, 0] = tg + jnp.where(k < NBLK, tgblk, 0.0)

    mskc = mskc_ref[0]
    e = e_ref[...]
    s = s_ref[...]
    P = [p_ref[c * HB:(c + 1) * HB, :] for c in range(NCH)]
    O = [o_ref[c * HB:(c + 1) * HB, :] for c in range(NCH)]
    for i in range(BS):
        exi = [ex_cons[i * BAT + c * HB:i * BAT + (c + 1) * HB, :]
               for c in range(NCH)]
        g = [jnp.dot(P[c], e, preferred_element_type=jnp.float32)
             for c in range(NCH)]
        ex = [g[c] * exi[c] for c in range(NCH)]
        pn = [jnp.dot(ex[c], s, preferred_element_type=jnp.float32)
              for c in range(NCH)]
        bcol = bnd_cons[:, i:i + 1]
        mcol = jax.lax.slice(mskc, (0, i), (BAT, i + 1)) > 0.0
        for c in range(NCH):
            lo = c * HB
            mc = jax.lax.slice(mcol, (lo, 0), (lo + HB, 1))
            bc = jax.lax.slice(bcol, (lo, 0), (lo + HB, 1))
            pnew = jnp.where(mc, pn[c], P[c])
            onew = jnp.where(mc, O[c] + bc, O[c])
            if i == 0:
                init_p = ex_cons[lo:lo + HB, START * T:START * T + T]
                pnew = jnp.where(k == 1, init_p, pnew)
                onew = jnp.where(k == 1, jnp.zeros_like(onew) + bc, onew)
            P[c], O[c] = pnew, onew
    for c in range(NCH):
        lo = c * HB
        pmax = jnp.maximum(jnp.max(P[c], axis=1, keepdims=True), 1e-30)
        p_ref[lo:lo + HB, :] = P[c] / pmax
        o_ref[lo:lo + HB, :] = O[c] + jnp.log(pmax)


def _crf_body(feats_ref, tgt_ref, mskp_ref, mskc_ref, wt_ref, b_ref, e_ref,
              s_ref, out_ref, ex_a, ex_b, bnd_a, bnd_b, p_ref, o_ref, tg_ref):
    k = pl.program_id(0)
    p = jax.lax.rem(k, 2)

    @pl.when(p == 0)
    def _():
        _phase(k, feats_ref, tgt_ref, mskp_ref, mskc_ref, wt_ref, b_ref,
               e_ref, s_ref, ex_a, bnd_a, ex_b, bnd_b, p_ref, o_ref, tg_ref)

    @pl.when(p == 1)
    def _():
        _phase(k, feats_ref, tgt_ref, mskp_ref, mskc_ref, wt_ref, b_ref,
               e_ref, s_ref, ex_b, bnd_b, ex_a, bnd_a, p_ref, o_ref, tg_ref)

    @pl.when(k == NBLK)
    def _():
        pend = jnp.log(jnp.maximum(p_ref[...], 1e-38)) + o_ref[...]
        logz = jnp.sum(jax.lax.slice(pend, (0, END), (BAT, END + 1)))
        out_ref[0, 0] = (logz - tg_ref[0, 0]) / float(BAT)


def kernel(feats, target, mask, W, b):
    feats2 = feats.reshape(SEQ * BAT, HID)
    wt = W.T.astype(jnp.bfloat16)
    b2 = b.reshape(1, TT)
    tgt = target[..., 0].astype(jnp.int32).reshape(NBLK, BS, BAT).transpose(0, 2, 1)
    msk = mask.astype(jnp.float32).reshape(NBLK, BS, BAT).transpose(0, 2, 1)
    jj = jnp.arange(TT, dtype=jnp.int32)
    e_mat = (jj[None, :] // T == jnp.arange(T, dtype=jnp.int32)[:, None]).astype(jnp.float32)
    s_mat = (jj[:, None] % T == jnp.arange(T, dtype=jnp.int32)[None, :]).astype(jnp.float32)

    out = pl.pallas_call(
        _crf_body,
        grid=(NBLK + 1,),
        in_specs=[
            pl.BlockSpec((ROWS, HID), lambda k: (jnp.minimum(k, NBLK - 1), 0)),
            pl.BlockSpec((1, BAT, BS), lambda k: (jnp.minimum(k, NBLK - 1), 0, 0)),
            pl.BlockSpec((1, BAT, BS), lambda k: (jnp.minimum(k, NBLK - 1), 0, 0)),
            pl.BlockSpec((1, BAT, BS), lambda k: (jnp.maximum(k - 1, 0), 0, 0)),
            pl.BlockSpec((HID, TT), lambda k: (0, 0)),
            pl.BlockSpec((1, TT), lambda k: (0, 0)),
            pl.BlockSpec((T, TT), lambda k: (0, 0)),
            pl.BlockSpec((TT, T), lambda k: (0, 0)),
        ],
        out_specs=pl.BlockSpec((1, 1), lambda k: (0, 0), memory_space=pltpu.SMEM),
        out_shape=jax.ShapeDtypeStruct((1, 1), jnp.float32),
        scratch_shapes=[
            pltpu.VMEM((ROWS, TT), jnp.float32),
            pltpu.VMEM((ROWS, TT), jnp.float32),
            pltpu.VMEM((BAT, BS), jnp.float32),
            pltpu.VMEM((BAT, BS), jnp.float32),
            pltpu.VMEM((BAT, T), jnp.float32),
            pltpu.VMEM((BAT, T), jnp.float32),
            pltpu.SMEM((1, 1), jnp.float32),
        ],
        compiler_params=pltpu.CompilerParams(dimension_semantics=("arbitrary",)),
    )(feats2, tgt, msk, msk, wt, b2, e_mat, s_mat)
    return out[0, 0]
